# Initial kernel scaffold; baseline (speedup 1.0000x reference)
import jax
import jax.numpy as jnp
from jax.experimental import pallas as pl


def kernel(G, c_3, u_3, v_3, t_tau, batch, num_graphs, Q3, K3, T_params, num_nodes):
    raise NotImplementedError("write your pallas kernel here")


# speedup vs baseline: 43.4472x; 43.4472x over previous
import math

import jax
import jax.numpy as jnp
from jax import lax
from jax.experimental import pallas as pl
from jax.experimental.pallas import tpu as pltpu
from jax.experimental.pallas import tpu_sc as plsc

D = 16
R = 1
N_NODES = 100000
N_MOTIFS = 1600000
NUM_TAU = 16
NUM_GRAPHS = 8

NC = 2
NS = 16
NW = NC * NS
B = 1024
NSUB = B // 128
CHUNKS = -(-N_MOTIFS // (NW * B))
M_PAD = NW * B * CHUNKS
GROUPS = B // 16

S_ACC = 100352
SLICE = S_ACC // NS
DUMP = N_NODES

LAMBDA_3 = math.log1p(math.exp(0.5))
BETA_3 = min(math.log1p(math.exp(1.0)), 5.0)
COEF = BETA_3 / math.sqrt(R * D)
OUT_SCALE = LAMBDA_3 / BETA_3


def _sc_body(c_hbm, u_hbm, v_hbm, t_hbm, q_hbm, k_hbm, tt_hbm, z_hbm,
             out_hbm, c_v, u_v, v_v, t_v, q_r, ku_r, kv_r, e_v, tt_v,
             acc_sp, sem):
    cid = lax.axis_index("c")
    sid = lax.axis_index("s")
    wid = sid * NC + cid

    pltpu.sync_copy(z_hbm, acc_sp.at[pl.ds(sid * SLICE, SLICE)])
    pltpu.sync_copy(tt_hbm, tt_v)
    plsc.subcore_barrier()

    lane = lax.iota(jnp.int32, 16)

    def chunk_body(ci, carry):
        row0 = (wid * CHUNKS + ci) * NSUB
        pltpu.sync_copy(c_hbm.at[pl.ds(row0, NSUB)], c_v)
        pltpu.sync_copy(u_hbm.at[pl.ds(row0, NSUB)], u_v)
        pltpu.sync_copy(v_hbm.at[pl.ds(row0, NSUB)], v_v)
        pltpu.sync_copy(t_hbm.at[pl.ds(row0 * 128, B)], t_v)

        cps = []
        for j in range(NSUB):
            dst = pl.ds(j * 128, 128)
            cps.append(pltpu.async_copy(q_hbm.at[c_v.at[j]], q_r.at[dst], sem))
            cps.append(pltpu.async_copy(k_hbm.at[u_v.at[j]], ku_r.at[dst], sem))
            cps.append(pltpu.async_copy(k_hbm.at[v_v.at[j]], kv_r.at[dst], sem))
        for cp in cps:
            cp.wait()

        def group_body(g, carry2):
            m0 = g * 16
            t16 = t_v[pl.ds(m0, 16)]
            bvec = jnp.zeros((16,), jnp.float32)
            for i in range(16):
                m = m0 + i
                trow = tt_v[t16[i], :]
                w = q_r[m, :] * (ku_r[m, :] * kv_r[m, :] + trow)
                s = jnp.sum(w)
                bvec = jnp.where(lane == i, s, bvec)
            e = jnp.exp(bvec * COEF)
            e_v[pl.ds(m0, 16)] = e
            return carry2

        lax.fori_loop(0, GROUPS, group_body, 0)

        for j in range(NSUB):
            pltpu.sync_copy(e_v.at[pl.ds(j * 128, 128)],
                            acc_sp.at[c_v.at[j]], add=True)
        return carry

    lax.fori_loop(0, CHUNKS, chunk_body, 0)

    plsc.subcore_barrier()
    pltpu.sync_copy(acc_sp.at[pl.ds(sid * SLICE, SLICE)],
                    out_hbm.at[cid, pl.ds(sid * SLICE, SLICE)])


def _tc_finish_body(s0_ref, s1_ref, b_ref, o_ref):
    s = s0_ref[...] + s1_ref[...]
    lse = jnp.where(s > 0.0, jnp.log(s), 0.0)
    for g in range(NUM_GRAPHS):
        eg = jnp.sum(jnp.where(b_ref[...] == g, lse, 0.0))
        o_ref[g] = eg * OUT_SCALE


def kernel(G, c_3, u_3, v_3, t_tau, batch, num_graphs, Q3, K3, T_params,
           num_nodes):
    del G, num_graphs, num_nodes
    pad = M_PAD - N_MOTIFS
    i32 = jnp.int32
    c_p = jnp.concatenate([c_3.astype(i32),
                           jnp.full((pad,), DUMP, i32)]).reshape(-1, 128)
    u_p = jnp.concatenate([u_3.astype(i32),
                           jnp.zeros((pad,), i32)]).reshape(-1, 128)
    v_p = jnp.concatenate([v_3.astype(i32),
                           jnp.zeros((pad,), i32)]).reshape(-1, 128)
    t_p = jnp.concatenate([t_tau.astype(i32), jnp.zeros((pad,), i32)])
    q2 = Q3.reshape(N_NODES, R * D)
    k2 = K3.reshape(N_NODES, R * D)
    tt = T_params.reshape(NUM_TAU, R * D)
    zeros = jnp.zeros((SLICE,), jnp.float32)

    mesh = plsc.VectorSubcoreMesh(core_axis_name="c", subcore_axis_name="s")
    sc = pl.kernel(
        _sc_body,
        out_type=jax.ShapeDtypeStruct((NC, S_ACC), jnp.float32),
        mesh=mesh,
        scratch_types=[
            pltpu.VMEM((NSUB, 128), i32),
            pltpu.VMEM((NSUB, 128), i32),
            pltpu.VMEM((NSUB, 128), i32),
            pltpu.VMEM((B,), i32),
            pltpu.VMEM((B, R * D), jnp.float32),
            pltpu.VMEM((B, R * D), jnp.float32),
            pltpu.VMEM((B, R * D), jnp.float32),
            pltpu.VMEM((B,), jnp.float32),
            pltpu.VMEM((NUM_TAU, R * D), jnp.float32),
            pltpu.VMEM_SHARED((S_ACC,), jnp.float32),
            pltpu.SemaphoreType.DMA,
        ],
        compiler_params=pltpu.CompilerParams(
            needs_layout_passes=False, use_tc_tiling_on_sc=False),
    )
    partials = sc(c_p, u_p, v_p, t_p, q2, k2, tt, zeros)

    batch_pad = jnp.concatenate(
        [batch.astype(i32), jnp.full((S_ACC - N_NODES,), NUM_GRAPHS, i32)]
    ).reshape(-1, 128)
    s0 = partials[0].reshape(-1, 128)
    s1 = partials[1].reshape(-1, 128)

    out = pl.pallas_call(
        _tc_finish_body,
        out_shape=jax.ShapeDtypeStruct((NUM_GRAPHS,), jnp.float32),
        out_specs=pl.BlockSpec(memory_space=pltpu.SMEM),
    )(s0, s1, batch_pad)
    return out
